# baseline (speedup 1.0000x reference)
import functools

import jax
import jax.numpy as jnp
from jax import lax
from jax.experimental import pallas as pl
from jax.experimental.pallas import tpu as pltpu
from jax.experimental.pallas import tpu_sc as plsc

DIM = 16
B = 16384
K = 20
ROW = 1 + DIM
NU = 1000000
NI = 100000

_NC = 2
_NS = 16
_NW = _NC * _NS
_BPW = B // _NW


def _sc_gather(ut, it, uid, iid):
    mesh = plsc.VectorSubcoreMesh(core_axis_name="c", subcore_axis_name="s")

    @functools.partial(
        pl.kernel,
        mesh=mesh,
        compiler_params=pltpu.CompilerParams(use_tc_tiling_on_sc=False),
        out_type=jax.ShapeDtypeStruct((B * 128,), jnp.float32),
        scratch_types=[
            pltpu.VMEM((_BPW,), jnp.int32),
            pltpu.VMEM((_BPW,), jnp.int32),
            pltpu.VMEM((ROW, _BPW), jnp.float32),
            pltpu.VMEM((ROW, _BPW), jnp.float32),
            pltpu.VMEM((ROW, _BPW), jnp.float32),
            pltpu.VMEM((ROW, _BPW), jnp.int32),
            pltpu.SemaphoreType.DMA,
            pltpu.SemaphoreType.DMA,
            pltpu.SemaphoreType.DMA,
        ],
    )
    def k(ut_h, it_h, uid_h, iid_h, out_h,
          uidx_v, iidx_v, urow_v, irow_v, sval_v, sidx_v, usem, isem, osem):
        wid = lax.axis_index("s") * _NC + lax.axis_index("c")
        base = wid * _BPW
        pltpu.sync_copy(uid_h.at[pl.ds(base, _BPW)], uidx_v)
        pltpu.sync_copy(iid_h.at[pl.ds(base, _BPW)], iidx_v)
        ucps = [
            pltpu.async_copy(ut_h.at[f].at[uidx_v], urow_v.at[f], usem)
            for f in range(ROW)
        ]
        icps = [
            pltpu.async_copy(it_h.at[f].at[iidx_v], irow_v.at[f], isem)
            for f in range(ROW)
        ]
        for cp in ucps:
            cp.wait()
        for cp in icps:
            cp.wait()

        iv = lax.iota(jnp.int32, 16)
        for c in range(_BPW // 16):
            sl = pl.ds(c * 16, 16)
            addr = (iv + (c * 16 + base)) * 128
            qs = (urow_v[0, sl] + irow_v[0, sl]) * (-2.0)
            for d in range(DIM):
                ud = urow_v[1 + d, sl]
                id_ = irow_v[1 + d, sl]
                sval_v[d, sl] = ud + id_
                sidx_v[d, sl] = addr + d
                qs = qs + ud * ud + id_ * id_
            sval_v[DIM, sl] = qs
            sidx_v[DIM, sl] = addr + DIM
        ocps = [
            pltpu.async_copy(sval_v.at[j], out_h.at[sidx_v.at[j]], osem)
            for j in range(ROW)
        ]
        for cp in ocps:
            cp.wait()

    return k(ut, it, uid, iid)


_BB = 2048


def _tc_body(pk_ref, age_ref, gen_ref, occ_ref, kind_ref, lab_ref,
             atab_ref, gtab_ref, otab_ref, ktab_ref, out_ref):
    f32 = jnp.float32
    pk = pk_ref[...]
    s = pk[:, 0:DIM]
    qs = pk[:, DIM:DIM + 1]
    bias = jnp.zeros((_BB, 1), f32)

    def one_hot_feature(idx_col, tab, width):
        t = lax.broadcasted_iota(jnp.int32, (_BB, width), 1)
        oh = (idx_col == t).astype(f32)
        return jnp.dot(oh, tab, precision=lax.Precision.HIGHEST,
                       preferred_element_type=f32)

    arow = one_hot_feature(age_ref[...], atab_ref[...], 8)
    grow = one_hot_feature(gen_ref[...], gtab_ref[...], 3)
    orow = one_hot_feature(occ_ref[...], otab_ref[...], 32)
    for row in (arow, grow, orow):
        bias = bias + row[:, 0:1]
        e = row[:, 1:ROW]
        s = s + e
        qs = qs + jnp.sum(e * e, axis=1, keepdims=True)

    kidx = kind_ref[...]
    t20 = lax.broadcasted_iota(jnp.int32, (_BB, K), 1)
    counts = jnp.zeros((_BB, K), f32)
    for k in range(K):
        counts = counts + (kidx[:, k:k + 1] == t20).astype(f32)
    counts = jnp.where(t20 != 0, counts, 0.0)
    ktab = ktab_ref[...]
    krow = jnp.dot(counts, ktab, precision=lax.Precision.HIGHEST,
                   preferred_element_type=f32)
    kq = jnp.dot(counts, jnp.sum(ktab[:, 1:ROW] * ktab[:, 1:ROW], axis=1,
                                 keepdims=True),
                 precision=lax.Precision.HIGHEST,
                 preferred_element_type=f32)
    bias = bias + krow[:, 0:1]
    s = s + krow[:, 1:ROW]
    qs = qs + kq

    two = 0.5 * (jnp.sum(s * s, axis=1, keepdims=True) - qs)
    logit = bias + two
    p = 1.0 / (1.0 + jnp.exp(-logit))
    lab = lab_ref[...]
    bce = -(lab * jnp.log(p + 1e-6) + (1.0 - lab) * jnp.log(1.0 - p + 1e-6))
    part = jnp.sum(bce) * (1.0 / B)

    @pl.when(pl.program_id(0) == 0)
    def _():
        out_ref[...] = jnp.zeros_like(out_ref)

    out_ref[...] = out_ref[...] + part


def _tc_loss(packed, age, gen, occ, kind, lab, atab, gtab, otab, ktab):
    grid = (B // _BB,)
    blk = lambda shape: pl.BlockSpec(shape, lambda i: (i, 0))
    rep = lambda shape: pl.BlockSpec(shape, lambda i: (0, 0))
    out = pl.pallas_call(
        _tc_body,
        grid=grid,
        in_specs=[
            blk((_BB, 128)),
            blk((_BB, 1)), blk((_BB, 1)), blk((_BB, 1)),
            blk((_BB, K)), blk((_BB, 1)),
            rep((8, ROW)), rep((3, ROW)), rep((32, ROW)), rep((20, ROW)),
        ],
        out_specs=rep((1, 1)),
        out_shape=jax.ShapeDtypeStruct((1, 1), jnp.float32),
    )(packed, age, gen, occ, kind, lab, atab, gtab, otab, ktab)
    return out[0, 0]


def kernel(userid, itemid, user_age, gender, user_occupation, item_kind,
           label, user_table, item_table, age_table, gender_table,
           occupation_table, kind_table):
    uid = userid.reshape(B).astype(jnp.int32)
    iid = itemid.reshape(B).astype(jnp.int32)
    uflat = jnp.ravel(user_table.T)
    iflat = jnp.ravel(item_table.T)
    packed1d = _sc_gather(uflat.reshape(ROW, NU), iflat.reshape(ROW, NI),
                          uid, iid)
    return _tc_loss(packed1d.reshape(B, 128), user_age, gender,
                    user_occupation, item_kind, label, age_table,
                    gender_table, occupation_table, kind_table)


# speedup vs baseline: 2.1382x; 2.1382x over previous
import functools

import jax
import jax.numpy as jnp
from jax import lax
from jax.experimental import pallas as pl
from jax.experimental.pallas import tpu as pltpu
from jax.experimental.pallas import tpu_sc as plsc

DIM = 16
B = 16384
K = 20
ROW = 1 + DIM
NU = 1000000
NI = 100000

_NC = 2
_NS = 16
_NW = _NC * _NS
_BPW = B // _NW
_CH = 256
_NCH = _BPW // _CH


def _sc_gather(up, ip, uid, iid):
    mesh = plsc.VectorSubcoreMesh(core_axis_name="c", subcore_axis_name="s")

    @functools.partial(
        pl.kernel,
        mesh=mesh,
        compiler_params=pltpu.CompilerParams(use_tc_tiling_on_sc=False),
        out_type=(
            jax.ShapeDtypeStruct((B, 128), jnp.float32),
            jax.ShapeDtypeStruct((B, 128), jnp.float32),
        ),
        scratch_types=[
            pltpu.VMEM((_NCH, _CH), jnp.int32),
            pltpu.VMEM((_NCH, _CH), jnp.int32),
            pltpu.VMEM((_CH, 128), jnp.float32),
            pltpu.VMEM((_CH, 128), jnp.float32),
            pltpu.SemaphoreType.DMA,
            pltpu.SemaphoreType.DMA,
        ],
    )
    def k(up_h, ip_h, uid_h, iid_h, ou_h, oi_h,
          uidx_v, iidx_v, ubuf_v, ibuf_v, usem, isem):
        wid = lax.axis_index("s") * _NC + lax.axis_index("c")
        base = wid * _BPW
        for c in range(_NCH):
            off = base + c * _CH
            pltpu.sync_copy(uid_h.at[pl.ds(off, _CH)], uidx_v.at[c])
            pltpu.sync_copy(iid_h.at[pl.ds(off, _CH)], iidx_v.at[c])
            ucp = pltpu.async_copy(up_h.at[uidx_v.at[c]], ubuf_v, usem)
            icp = pltpu.async_copy(ip_h.at[iidx_v.at[c]], ibuf_v, isem)
            ucp.wait()
            pltpu.sync_copy(ubuf_v, ou_h.at[pl.ds(off, _CH)])
            icp.wait()
            pltpu.sync_copy(ibuf_v, oi_h.at[pl.ds(off, _CH)])

    return k(up, ip, uid, iid)


_BB = 2048


def _tc_body(u_ref, i_ref, age_ref, gen_ref, occ_ref, kind_ref, lab_ref,
             atab_ref, gtab_ref, otab_ref, ktab_ref, out_ref):
    f32 = jnp.float32
    u = u_ref[...]
    i = i_ref[...]
    ue = u[:, 1:ROW]
    ie = i[:, 1:ROW]
    s = ue + ie
    qs = jnp.sum(ue * ue + ie * ie, axis=1, keepdims=True)
    bias = u[:, 0:1] + i[:, 0:1]

    def one_hot_feature(idx_col, tab, width):
        t = lax.broadcasted_iota(jnp.int32, (_BB, width), 1)
        oh = (idx_col == t).astype(f32)
        return jnp.dot(oh, tab, precision=lax.Precision.HIGHEST,
                       preferred_element_type=f32)

    arow = one_hot_feature(age_ref[...], atab_ref[...], 8)
    grow = one_hot_feature(gen_ref[...], gtab_ref[...], 3)
    orow = one_hot_feature(occ_ref[...], otab_ref[...], 32)
    for row in (arow, grow, orow):
        bias = bias + row[:, 0:1]
        e = row[:, 1:ROW]
        s = s + e
        qs = qs + jnp.sum(e * e, axis=1, keepdims=True)

    kidx = kind_ref[...]
    t20 = lax.broadcasted_iota(jnp.int32, (_BB, K), 1)
    counts = jnp.zeros((_BB, K), f32)
    for k in range(K):
        counts = counts + (kidx[:, k:k + 1] == t20).astype(f32)
    counts = jnp.where(t20 != 0, counts, 0.0)
    ktab = ktab_ref[...]
    krow = jnp.dot(counts, ktab, precision=lax.Precision.HIGHEST,
                   preferred_element_type=f32)
    kq = jnp.dot(counts, jnp.sum(ktab[:, 1:ROW] * ktab[:, 1:ROW], axis=1,
                                 keepdims=True),
                 precision=lax.Precision.HIGHEST,
                 preferred_element_type=f32)
    bias = bias + krow[:, 0:1]
    s = s + krow[:, 1:ROW]
    qs = qs + kq

    two = 0.5 * (jnp.sum(s * s, axis=1, keepdims=True) - qs)
    logit = bias + two
    p = 1.0 / (1.0 + jnp.exp(-logit))
    lab = lab_ref[...]
    bce = -(lab * jnp.log(p + 1e-6) + (1.0 - lab) * jnp.log(1.0 - p + 1e-6))
    part = jnp.sum(bce) * (1.0 / B)

    @pl.when(pl.program_id(0) == 0)
    def _():
        out_ref[...] = jnp.zeros_like(out_ref)

    out_ref[...] = out_ref[...] + part


def _tc_loss(urows, irows, age, gen, occ, kind, lab, atab, gtab, otab, ktab):
    grid = (B // _BB,)
    blk = lambda shape: pl.BlockSpec(shape, lambda i: (i, 0))
    rep = lambda shape: pl.BlockSpec(shape, lambda i: (0, 0))
    out = pl.pallas_call(
        _tc_body,
        grid=grid,
        in_specs=[
            blk((_BB, 128)), blk((_BB, 128)),
            blk((_BB, 1)), blk((_BB, 1)), blk((_BB, 1)),
            blk((_BB, K)), blk((_BB, 1)),
            rep((8, ROW)), rep((3, ROW)), rep((32, ROW)), rep((20, ROW)),
        ],
        out_specs=rep((1, 1)),
        out_shape=jax.ShapeDtypeStruct((1, 1), jnp.float32),
    )(urows, irows, age, gen, occ, kind, lab, atab, gtab, otab, ktab)
    return out[0, 0]


def kernel(userid, itemid, user_age, gender, user_occupation, item_kind,
           label, user_table, item_table, age_table, gender_table,
           occupation_table, kind_table):
    uid = userid.reshape(B).astype(jnp.int32)
    iid = itemid.reshape(B).astype(jnp.int32)
    up = jnp.pad(user_table, ((0, 0), (0, 128 - ROW)))
    ipad = jnp.pad(item_table, ((0, 0), (0, 128 - ROW)))
    urows, irows = _sc_gather(up, ipad, uid, iid)
    return _tc_loss(urows, irows, user_age, gender, user_occupation,
                    item_kind, label, age_table, gender_table,
                    occupation_table, kind_table)
